# Initial kernel scaffold; baseline (speedup 1.0000x reference)
import jax
import jax.numpy as jnp
from jax.experimental import pallas as pl


def kernel(mlmm_distances, atomic_charges, mlmm_atomic_charges, mlmm_idxu, mlmm_idxv, mlmm_vectors, atomic_dipoles):
    raise NotImplementedError("write your pallas kernel here")


# speedup vs baseline: 15.6622x; 15.6622x over previous
import jax
import jax.numpy as jnp
from jax import lax
from jax.experimental import pallas as pl
from jax.experimental.pallas import tpu as pltpu
from jax.experimental.pallas import tpu_sc as plsc

E = 6_400_000
N_ML = 100_000
N_MM = 100_000

NC = 2
NS = 16
LANES = 16
NW = NC * NS
EPT = E // NW
CHUNK = 4000
NCHUNK = EPT // CHUNK
GROUPS = CHUNK // LANES

ROWS_PER_SUB = 6256
PAD = ROWS_PER_SUB * NS

CUTOFF = 12.0
KE = 332.0637
CUTON = 9.0
CHI_SHIFT = 1.0 / CUTOFF
CHI2_SHIFT = CHI_SHIFT * CHI_SHIFT
INV_W = 1.0 / (CUTOFF - CUTON)


def _sc_body(d_hbm, idxu_hbm, idxv_hbm, vec_hbm, q_hbm, dx_hbm, dy_hbm,
             dz_hbm, qmm_hbm, out_hbm,
             q_s, dx_s, dy_s, dz_s, qmm_s,
             d_v, idxu_v, idxv_v, vec_v, qi_v, dxg_v, dyg_v, dzg_v, qj_v,
             out_v, sem0, sem1):
    c = lax.axis_index("c")
    s = lax.axis_index("s")
    wid = s * NC + c

    r0 = s * ROWS_PER_SUB
    sl = pl.ds(r0, ROWS_PER_SUB)
    bounce = vec_v.at[pl.ds(0, ROWS_PER_SUB)]
    for hbm_t, sp_t in ((q_hbm, q_s), (dx_hbm, dx_s), (dy_hbm, dy_s),
                        (dz_hbm, dz_s), (qmm_hbm, qmm_s)):
        pltpu.sync_copy(hbm_t.at[sl], bounce)
        pltpu.sync_copy(bounce, sp_t.at[sl])
    plsc.subcore_barrier()

    lane = lax.iota(jnp.int32, LANES)
    perm = {}
    sel0 = {}
    sel1 = {}
    for comp in range(3):
        flat = lane * 3 + comp
        perm[comp] = jnp.bitwise_and(flat, LANES - 1)
        src = lax.shift_right_logical(flat, 4)
        sel0[comp] = src == 0
        sel1[comp] = src == 1

    gd = lax.GatherDimensionNumbers(
        offset_dims=(), collapsed_slice_dims=(0,), start_index_map=(0,))

    def reg_perm(v, idx):
        return lax.gather(v, idx[:, None], gd, slice_sizes=(1,),
                          mode=lax.GatherScatterMode.PROMISE_IN_BOUNDS)

    def deinterleave(v0, v1, v2, comp):
        a = reg_perm(v0, perm[comp])
        b = reg_perm(v1, perm[comp])
        cc = reg_perm(v2, perm[comp])
        return jnp.where(sel0[comp], a, jnp.where(sel1[comp], b, cc))

    def chunk_body(k, carry):
        base = wid * EPT + k * CHUNK
        pltpu.sync_copy(d_hbm.at[pl.ds(base, CHUNK)], d_v)
        pltpu.sync_copy(idxu_hbm.at[pl.ds(base, CHUNK)], idxu_v)
        pltpu.sync_copy(idxv_hbm.at[pl.ds(base, CHUNK)], idxv_v)
        pltpu.sync_copy(vec_hbm.at[pl.ds(3 * base, 3 * CHUNK)], vec_v)
        cps = [
            pltpu.async_copy(q_s.at[idxu_v], qi_v, sem0),
            pltpu.async_copy(dx_s.at[idxu_v], dxg_v, sem0),
            pltpu.async_copy(dy_s.at[idxu_v], dyg_v, sem0),
            pltpu.async_copy(dz_s.at[idxu_v], dzg_v, sem0),
            pltpu.async_copy(qmm_s.at[idxv_v], qj_v, sem1),
        ]
        for cp in cps:
            cp.wait()

        def group_body(g, carry2):
            o = g * LANES
            d = d_v[pl.ds(o, LANES)]
            qj = qj_v[pl.ds(o, LANES)]
            qi = qi_v[pl.ds(o, LANES)]
            dx = dxg_v[pl.ds(o, LANES)]
            dy = dyg_v[pl.ds(o, LANES)]
            dz = dzg_v[pl.ds(o, LANES)]
            o3 = o * 3
            v0 = vec_v[pl.ds(o3, LANES)]
            v1 = vec_v[pl.ds(o3 + LANES, LANES)]
            v2 = vec_v[pl.ds(o3 + 2 * LANES, LANES)]
            vx = deinterleave(v0, v1, v2, 0)
            vy = deinterleave(v0, v1, v2, 1)
            vz = deinterleave(v0, v1, v2, 2)

            chi = 1.0 / d
            e1 = qi * qj * (chi - CHI_SHIFT)
            dot = (vx * dx + vy * dy + vz * dz) * chi
            e2 = qj * dot * (chi * chi - CHI2_SHIFT)
            x = jnp.clip((d - CUTON) * INV_W, 0.0, 1.0)
            sw = 1.0 - x * x * x * (x * (6.0 * x - 15.0) + 10.0)
            out_v[pl.ds(o, LANES)] = KE * (e1 + e2) * sw
            return carry2

        lax.fori_loop(0, GROUPS, group_body, 0)
        pltpu.sync_copy(out_v, out_hbm.at[pl.ds(base, CHUNK)])
        return carry

    lax.fori_loop(0, NCHUNK, chunk_body, 0)


def kernel(mlmm_distances, atomic_charges, mlmm_atomic_charges,
           mlmm_idxu, mlmm_idxv, mlmm_vectors, atomic_dipoles):
    padn = PAD - N_ML
    q = jnp.pad(atomic_charges, (0, padn))
    dx = jnp.pad(atomic_dipoles[:, 0], (0, padn))
    dy = jnp.pad(atomic_dipoles[:, 1], (0, padn))
    dz = jnp.pad(atomic_dipoles[:, 2], (0, padn))
    qmm = jnp.pad(mlmm_atomic_charges, (0, PAD - N_MM))
    vec_flat = mlmm_vectors.reshape(-1)

    f = pl.kernel(
        _sc_body,
        out_type=jax.ShapeDtypeStruct((E,), jnp.float32),
        mesh=plsc.VectorSubcoreMesh(core_axis_name="c", subcore_axis_name="s"),
        scratch_types=[
            pltpu.VMEM_SHARED((PAD,), jnp.float32),
            pltpu.VMEM_SHARED((PAD,), jnp.float32),
            pltpu.VMEM_SHARED((PAD,), jnp.float32),
            pltpu.VMEM_SHARED((PAD,), jnp.float32),
            pltpu.VMEM_SHARED((PAD,), jnp.float32),
            pltpu.VMEM((CHUNK,), jnp.float32),
            pltpu.VMEM((CHUNK,), jnp.int32),
            pltpu.VMEM((CHUNK,), jnp.int32),
            pltpu.VMEM((3 * CHUNK,), jnp.float32),
            pltpu.VMEM((CHUNK,), jnp.float32),
            pltpu.VMEM((CHUNK,), jnp.float32),
            pltpu.VMEM((CHUNK,), jnp.float32),
            pltpu.VMEM((CHUNK,), jnp.float32),
            pltpu.VMEM((CHUNK,), jnp.float32),
            pltpu.VMEM((CHUNK,), jnp.float32),
            pltpu.SemaphoreType.DMA,
            pltpu.SemaphoreType.DMA,
        ],
    )
    return f(mlmm_distances, mlmm_idxu, mlmm_idxv, vec_flat, q, dx, dy, dz,
             qmm)


# speedup vs baseline: 188.7833x; 12.0535x over previous
import jax
import jax.numpy as jnp
from jax import lax
from jax.experimental import pallas as pl
from jax.experimental.pallas import tpu as pltpu
from jax.experimental.pallas import tpu_sc as plsc

E = 6_400_000
N_ML = 100_000
N_MM = 100_000

NC = 2
NS = 16
LANES = 16
NW = NC * NS
EPT = E // NW
CHUNK = 4000
NCHUNK = EPT // CHUNK
GROUPS = CHUNK // LANES

ROWS_PER_SUB = 6256
PAD = ROWS_PER_SUB * NS

CUTOFF = 12.0
KE = 332.0637
CUTON = 9.0
CHI_SHIFT = 1.0 / CUTOFF
CHI2_SHIFT = CHI_SHIFT * CHI_SHIFT
INV_W = 1.0 / (CUTOFF - CUTON)


def _sc_body(d_hbm, idxu_hbm, idxv_hbm, vx_hbm, vy_hbm, vz_hbm,
             q_hbm, dx_hbm, dy_hbm, dz_hbm, qmm_hbm, out_hbm,
             q_s, dx_s, dy_s, dz_s, qmm_s,
             bounce_v, d_v, idxu_v, idxv_v, vx_v, vy_v, vz_v,
             qi_v, dxg_v, dyg_v, dzg_v, qj_v, out_v,
             sem0, sem1):
    c = lax.axis_index("c")
    s = lax.axis_index("s")
    wid = s * NC + c

    sl = pl.ds(s * ROWS_PER_SUB, ROWS_PER_SUB)
    for hbm_t, sp_t in ((q_hbm, q_s), (dx_hbm, dx_s), (dy_hbm, dy_s),
                        (dz_hbm, dz_s), (qmm_hbm, qmm_s)):
        pltpu.sync_copy(hbm_t.at[sl], bounce_v)
        pltpu.sync_copy(bounce_v, sp_t.at[sl])
    plsc.subcore_barrier()

    def chunk_body(k, carry):
        base = wid * EPT + k * CHUNK
        ces = pl.ds(base, CHUNK)
        pltpu.sync_copy(d_hbm.at[ces], d_v)
        pltpu.sync_copy(idxu_hbm.at[ces], idxu_v)
        pltpu.sync_copy(idxv_hbm.at[ces], idxv_v)
        pltpu.sync_copy(vx_hbm.at[ces], vx_v)
        pltpu.sync_copy(vy_hbm.at[ces], vy_v)
        pltpu.sync_copy(vz_hbm.at[ces], vz_v)
        cps = [
            pltpu.async_copy(q_s.at[idxu_v], qi_v, sem0),
            pltpu.async_copy(dx_s.at[idxu_v], dxg_v, sem0),
            pltpu.async_copy(dy_s.at[idxu_v], dyg_v, sem0),
            pltpu.async_copy(dz_s.at[idxu_v], dzg_v, sem0),
            pltpu.async_copy(qmm_s.at[idxv_v], qj_v, sem1),
        ]
        for cp in cps:
            cp.wait()

        def group_body(g, carry2):
            o = g * LANES
            osl = pl.ds(o, LANES)
            d = d_v[osl]
            qj = qj_v[osl]
            qi = qi_v[osl]
            dx = dxg_v[osl]
            dy = dyg_v[osl]
            dz = dzg_v[osl]
            vx = vx_v[osl]
            vy = vy_v[osl]
            vz = vz_v[osl]

            chi = 1.0 / d
            e1 = qi * qj * (chi - CHI_SHIFT)
            dot = (vx * dx + vy * dy + vz * dz) * chi
            e2 = qj * dot * (chi * chi - CHI2_SHIFT)
            x = jnp.clip((d - CUTON) * INV_W, 0.0, 1.0)
            sw = 1.0 - x * x * x * (x * (6.0 * x - 15.0) + 10.0)
            out_v[osl] = KE * (e1 + e2) * sw
            return carry2

        lax.fori_loop(0, GROUPS, group_body, 0)
        pltpu.sync_copy(out_v, out_hbm.at[ces])
        return carry

    lax.fori_loop(0, NCHUNK, chunk_body, 0)


def kernel(mlmm_distances, atomic_charges, mlmm_atomic_charges,
           mlmm_idxu, mlmm_idxv, mlmm_vectors, atomic_dipoles):
    padn = PAD - N_ML
    q = jnp.pad(atomic_charges, (0, padn))
    dx = jnp.pad(atomic_dipoles[:, 0], (0, padn))
    dy = jnp.pad(atomic_dipoles[:, 1], (0, padn))
    dz = jnp.pad(atomic_dipoles[:, 2], (0, padn))
    qmm = jnp.pad(mlmm_atomic_charges, (0, PAD - N_MM))
    vx = mlmm_vectors[:, 0]
    vy = mlmm_vectors[:, 1]
    vz = mlmm_vectors[:, 2]

    f = pl.kernel(
        _sc_body,
        out_type=jax.ShapeDtypeStruct((E,), jnp.float32),
        mesh=plsc.VectorSubcoreMesh(core_axis_name="c", subcore_axis_name="s"),
        scratch_types=[
            pltpu.VMEM_SHARED((PAD,), jnp.float32),
            pltpu.VMEM_SHARED((PAD,), jnp.float32),
            pltpu.VMEM_SHARED((PAD,), jnp.float32),
            pltpu.VMEM_SHARED((PAD,), jnp.float32),
            pltpu.VMEM_SHARED((PAD,), jnp.float32),
            pltpu.VMEM((ROWS_PER_SUB,), jnp.float32),
            pltpu.VMEM((CHUNK,), jnp.float32),
            pltpu.VMEM((CHUNK,), jnp.int32),
            pltpu.VMEM((CHUNK,), jnp.int32),
            pltpu.VMEM((CHUNK,), jnp.float32),
            pltpu.VMEM((CHUNK,), jnp.float32),
            pltpu.VMEM((CHUNK,), jnp.float32),
            pltpu.VMEM((CHUNK,), jnp.float32),
            pltpu.VMEM((CHUNK,), jnp.float32),
            pltpu.VMEM((CHUNK,), jnp.float32),
            pltpu.VMEM((CHUNK,), jnp.float32),
            pltpu.VMEM((CHUNK,), jnp.float32),
            pltpu.VMEM((CHUNK,), jnp.float32),
            pltpu.SemaphoreType.DMA,
            pltpu.SemaphoreType.DMA,
        ],
    )
    return f(mlmm_distances, mlmm_idxu, mlmm_idxv, vx, vy, vz,
             q, dx, dy, dz, qmm)


# speedup vs baseline: 314.1708x; 1.6642x over previous
import jax
import jax.numpy as jnp
from jax import lax
from jax.experimental import pallas as pl
from jax.experimental.pallas import tpu as pltpu
from jax.experimental.pallas import tpu_sc as plsc

E = 6_400_000
N_ML = 100_000
N_MM = 100_000

NC = 2
NS = 16
LANES = 16
NW = NC * NS
EPT = E // NW
CHUNK = 4000
NCHUNK = EPT // CHUNK
NPAIR = NCHUNK // 2
GROUPS = CHUNK // LANES

ROWS_PER_SUB = 6256
PAD = ROWS_PER_SUB * NS

CUTOFF = 12.0
KE = 332.0637
CUTON = 9.0
CHI_SHIFT = 1.0 / CUTOFF
CHI2_SHIFT = CHI_SHIFT * CHI_SHIFT
INV_W = 1.0 / (CUTOFF - CUTON)


def _sc_body(d_hbm, idxu_hbm, idxv_hbm, vx_hbm, vy_hbm, vz_hbm,
             q_hbm, dx_hbm, dy_hbm, dz_hbm, qmm_hbm, out_hbm,
             q_s, dx_s, dy_s, dz_s, qmm_s,
             d_v, idxu_v, idxv_v, vx_v, vy_v, vz_v,
             qi_v, dxg_v, dyg_v, dzg_v, qj_v, out_v,
             sem_in0, sem_in1, sem_g0, sem_g1, sem_o0, sem_o1):
    c = lax.axis_index("c")
    s = lax.axis_index("s")
    wid = s * NC + c
    tile_base = wid * EPT

    sl = pl.ds(s * ROWS_PER_SUB, ROWS_PER_SUB)
    bounce_v = vx_v.at[pl.ds(0, ROWS_PER_SUB)]
    for hbm_t, sp_t in ((q_hbm, q_s), (dx_hbm, dx_s), (dy_hbm, dy_s),
                        (dz_hbm, dz_s), (qmm_hbm, qmm_s)):
        pltpu.sync_copy(hbm_t.at[sl], bounce_v)
        pltpu.sync_copy(bounce_v, sp_t.at[sl])
    plsc.subcore_barrier()

    ins = (d_hbm, idxu_hbm, idxv_hbm, vx_hbm, vy_hbm, vz_hbm)
    inbufs = (d_v, idxu_v, idxv_v, vx_v, vy_v, vz_v)
    sem_in = (sem_in0, sem_in1)
    sem_g = (sem_g0, sem_g1)
    sem_o = (sem_o0, sem_o1)

    def ces(k):
        return pl.ds(tile_base + k * CHUNK, CHUNK)

    def half(buf, p):
        return buf.at[pl.ds(p * CHUNK, CHUNK)]

    def instream(k, p):
        sli = ces(k)
        for hbm_t, buf in zip(ins, inbufs):
            pltpu.async_copy(hbm_t.at[sli], half(buf, p), sem_in[p])

    def instream_wait(k, p):
        sli = ces(k)
        for hbm_t, buf in zip(ins, inbufs):
            pltpu.make_async_copy(hbm_t.at[sli], half(buf, p), sem_in[p]).wait()

    def gather(p):
        pltpu.async_copy(q_s.at[half(idxu_v, p)], half(qi_v, p), sem_g[p])
        pltpu.async_copy(dx_s.at[half(idxu_v, p)], half(dxg_v, p), sem_g[p])
        pltpu.async_copy(dy_s.at[half(idxu_v, p)], half(dyg_v, p), sem_g[p])
        pltpu.async_copy(dz_s.at[half(idxu_v, p)], half(dzg_v, p), sem_g[p])
        pltpu.async_copy(qmm_s.at[half(idxv_v, p)], half(qj_v, p), sem_g[p])

    def gather_wait(p):
        pltpu.make_async_copy(q_s.at[half(idxu_v, p)], half(qi_v, p),
                              sem_g[p]).wait()
        pltpu.make_async_copy(dx_s.at[half(idxu_v, p)], half(dxg_v, p),
                              sem_g[p]).wait()
        pltpu.make_async_copy(dy_s.at[half(idxu_v, p)], half(dyg_v, p),
                              sem_g[p]).wait()
        pltpu.make_async_copy(dz_s.at[half(idxu_v, p)], half(dzg_v, p),
                              sem_g[p]).wait()
        pltpu.make_async_copy(qmm_s.at[half(idxv_v, p)], half(qj_v, p),
                              sem_g[p]).wait()

    def outstream(k, p):
        pltpu.async_copy(half(out_v, p), out_hbm.at[ces(k)], sem_o[p])

    def outstream_wait(k, p):
        pltpu.make_async_copy(half(out_v, p), out_hbm.at[ces(k)],
                              sem_o[p]).wait()

    def compute(p):
        dp, qjp, qip = half(d_v, p), half(qj_v, p), half(qi_v, p)
        dxp, dyp, dzp = half(dxg_v, p), half(dyg_v, p), half(dzg_v, p)
        vxp, vyp, vzp = half(vx_v, p), half(vy_v, p), half(vz_v, p)
        outp = half(out_v, p)

        def group_body(g, carry2):
            o = g * LANES
            osl = pl.ds(o, LANES)
            d = dp[osl]
            qj = qjp[osl]
            qi = qip[osl]
            dx = dxp[osl]
            dy = dyp[osl]
            dz = dzp[osl]
            vx = vxp[osl]
            vy = vyp[osl]
            vz = vzp[osl]

            chi = 1.0 / d
            e1 = qi * qj * (chi - CHI_SHIFT)
            dot = (vx * dx + vy * dy + vz * dz) * chi
            e2 = qj * dot * (chi * chi - CHI2_SHIFT)
            x = jnp.clip((d - CUTON) * INV_W, 0.0, 1.0)
            sw = 1.0 - x * x * x * (x * (6.0 * x - 15.0) + 10.0)
            outp[osl] = KE * (e1 + e2) * sw
            return carry2

        lax.fori_loop(0, GROUPS, group_body, 0)

    def halfstep(k, p, first_pair, last_pair):
        gather_wait(p)
        if not last_pair or p == 0:
            instream_wait(k + 1, 1 - p)
            gather(1 - p)
        if not first_pair:
            outstream_wait(k - 2, p)
        compute(p)
        outstream(k, p)
        if not last_pair:
            instream(k + 2, p)

    instream(0, 0)
    instream_wait(0, 0)
    gather(0)
    instream(1, 1)

    halfstep(0, 0, True, False)
    halfstep(1, 1, True, False)

    def pair_body(kp, carry):
        k0 = 2 * kp
        halfstep(k0, 0, False, False)
        halfstep(k0 + 1, 1, False, False)
        return carry

    lax.fori_loop(1, NPAIR - 1, pair_body, 0)

    k0 = NCHUNK - 2
    halfstep(k0, 0, False, True)
    halfstep(k0 + 1, 1, False, True)

    outstream_wait(NCHUNK - 2, 0)
    outstream_wait(NCHUNK - 1, 1)


def kernel(mlmm_distances, atomic_charges, mlmm_atomic_charges,
           mlmm_idxu, mlmm_idxv, mlmm_vectors, atomic_dipoles):
    padn = PAD - N_ML
    q = jnp.pad(atomic_charges, (0, padn))
    dx = jnp.pad(atomic_dipoles[:, 0], (0, padn))
    dy = jnp.pad(atomic_dipoles[:, 1], (0, padn))
    dz = jnp.pad(atomic_dipoles[:, 2], (0, padn))
    qmm = jnp.pad(mlmm_atomic_charges, (0, PAD - N_MM))
    vx = mlmm_vectors[:, 0]
    vy = mlmm_vectors[:, 1]
    vz = mlmm_vectors[:, 2]

    def dbuf(dt=jnp.float32):
        return pltpu.VMEM((2 * CHUNK,), dt)

    f = pl.kernel(
        _sc_body,
        out_type=jax.ShapeDtypeStruct((E,), jnp.float32),
        mesh=plsc.VectorSubcoreMesh(core_axis_name="c", subcore_axis_name="s"),
        scratch_types=[
            pltpu.VMEM_SHARED((PAD,), jnp.float32),
            pltpu.VMEM_SHARED((PAD,), jnp.float32),
            pltpu.VMEM_SHARED((PAD,), jnp.float32),
            pltpu.VMEM_SHARED((PAD,), jnp.float32),
            pltpu.VMEM_SHARED((PAD,), jnp.float32),
            dbuf(), dbuf(jnp.int32), dbuf(jnp.int32), dbuf(), dbuf(), dbuf(),
            dbuf(), dbuf(), dbuf(), dbuf(), dbuf(),
            dbuf(),
            pltpu.SemaphoreType.DMA, pltpu.SemaphoreType.DMA,
            pltpu.SemaphoreType.DMA, pltpu.SemaphoreType.DMA,
            pltpu.SemaphoreType.DMA, pltpu.SemaphoreType.DMA,
        ],
    )
    return f(mlmm_distances, mlmm_idxu, mlmm_idxv, vx, vy, vz,
             q, dx, dy, dz, qmm)


# speedup vs baseline: 428.4005x; 1.3636x over previous
import jax
import jax.numpy as jnp
from jax import lax
from jax.experimental import pallas as pl
from jax.experimental.pallas import tpu as pltpu
from jax.experimental.pallas import tpu_sc as plsc

E = 6_400_000
N_ML = 100_000
N_MM = 100_000

NC = 2
NS = 16
LANES = 16
NW = NC * NS
EPT = E // NW
CHUNK = 4000
NCHUNK = EPT // CHUNK
NPAIR = NCHUNK // 2
GROUPS = CHUNK // LANES

ROWS_PER_SUB = 6256
PAD = ROWS_PER_SUB * NS

CUTOFF = 12.0
KE = 332.0637
CUTON = 9.0
CHI_SHIFT = 1.0 / CUTOFF
CHI2_SHIFT = CHI_SHIFT * CHI_SHIFT
INV_W = 1.0 / (CUTOFF - CUTON)


def _sc_body(d_hbm, idxu_hbm, idxv_hbm, vx_hbm, vy_hbm, vz_hbm,
             qdx_hbm, dydz_hbm, qmm_hbm, out_hbm,
             qdx_s, dydz_s, qmm_s,
             d_v, idxu_v, idxv_v, vx_v, vy_v, vz_v,
             g1_v, g2_v, qj_v, out_v,
             sem_in0, sem_in1, sem_g0, sem_g1, sem_o0, sem_o1):
    c = lax.axis_index("c")
    s = lax.axis_index("s")
    wid = s * NC + c
    tile_base = wid * EPT

    sl = pl.ds(s * ROWS_PER_SUB, ROWS_PER_SUB)
    bounce_f = vx_v.at[pl.ds(0, ROWS_PER_SUB)]
    pltpu.sync_copy(qmm_hbm.at[sl], bounce_f)
    pltpu.sync_copy(bounce_f, qmm_s.at[sl])
    bounce_i = idxu_v.at[pl.ds(0, ROWS_PER_SUB)]
    for hbm_t, sp_t in ((qdx_hbm, qdx_s), (dydz_hbm, dydz_s)):
        pltpu.sync_copy(hbm_t.at[sl], bounce_i)
        pltpu.sync_copy(bounce_i, sp_t.at[sl])
    plsc.subcore_barrier()

    ins = (d_hbm, idxu_hbm, idxv_hbm, vx_hbm, vy_hbm, vz_hbm)
    inbufs = (d_v, idxu_v, idxv_v, vx_v, vy_v, vz_v)
    sem_in = (sem_in0, sem_in1)
    sem_g = (sem_g0, sem_g1)
    sem_o = (sem_o0, sem_o1)

    def ces(k):
        return pl.ds(tile_base + k * CHUNK, CHUNK)

    def half(buf, p):
        return buf.at[pl.ds(p * CHUNK, CHUNK)]

    def instream(k, p):
        sli = ces(k)
        for hbm_t, buf in zip(ins, inbufs):
            pltpu.async_copy(hbm_t.at[sli], half(buf, p), sem_in[p])

    def instream_wait(k, p):
        sli = ces(k)
        for hbm_t, buf in zip(ins, inbufs):
            pltpu.make_async_copy(hbm_t.at[sli], half(buf, p), sem_in[p]).wait()

    def gather(p):
        pltpu.async_copy(qdx_s.at[half(idxu_v, p)], half(g1_v, p), sem_g[p])
        pltpu.async_copy(dydz_s.at[half(idxu_v, p)], half(g2_v, p), sem_g[p])
        pltpu.async_copy(qmm_s.at[half(idxv_v, p)], half(qj_v, p), sem_g[p])

    def gather_wait(p):
        pltpu.make_async_copy(qdx_s.at[half(idxu_v, p)], half(g1_v, p),
                              sem_g[p]).wait()
        pltpu.make_async_copy(dydz_s.at[half(idxu_v, p)], half(g2_v, p),
                              sem_g[p]).wait()
        pltpu.make_async_copy(qmm_s.at[half(idxv_v, p)], half(qj_v, p),
                              sem_g[p]).wait()

    def outstream(k, p):
        pltpu.async_copy(half(out_v, p), out_hbm.at[ces(k)], sem_o[p])

    def outstream_wait(k, p):
        pltpu.make_async_copy(half(out_v, p), out_hbm.at[ces(k)],
                              sem_o[p]).wait()

    def compute(p):
        dp, qjp = half(d_v, p), half(qj_v, p)
        g1p, g2p = half(g1_v, p), half(g2_v, p)
        vxp, vyp, vzp = half(vx_v, p), half(vy_v, p), half(vz_v, p)
        outp = half(out_v, p)

        def group_body(g, carry2):
            o = g * LANES
            osl = pl.ds(o, LANES)
            d = dp[osl]
            qj = qjp[osl]
            w1 = g1p[osl]
            w2 = g2p[osl]
            qi = lax.bitcast_convert_type(lax.shift_left(w1, 16), jnp.float32)
            dx = lax.bitcast_convert_type(jnp.bitwise_and(w1, -65536), jnp.float32)
            dy = lax.bitcast_convert_type(lax.shift_left(w2, 16), jnp.float32)
            dz = lax.bitcast_convert_type(jnp.bitwise_and(w2, -65536), jnp.float32)
            vx = vxp[osl]
            vy = vyp[osl]
            vz = vzp[osl]

            chi = 1.0 / d
            e1 = qi * qj * (chi - CHI_SHIFT)
            dot = (vx * dx + vy * dy + vz * dz) * chi
            e2 = qj * dot * (chi * chi - CHI2_SHIFT)
            x = jnp.clip((d - CUTON) * INV_W, 0.0, 1.0)
            sw = 1.0 - x * x * x * (x * (6.0 * x - 15.0) + 10.0)
            outp[osl] = KE * (e1 + e2) * sw
            return carry2

        lax.fori_loop(0, GROUPS, group_body, 0)

    def halfstep(k, p, first_pair, last_pair):
        gather_wait(p)
        if not last_pair or p == 0:
            instream_wait(k + 1, 1 - p)
            gather(1 - p)
        if not first_pair:
            outstream_wait(k - 2, p)
        compute(p)
        outstream(k, p)
        if not last_pair:
            instream(k + 2, p)

    instream(0, 0)
    instream_wait(0, 0)
    gather(0)
    instream(1, 1)

    halfstep(0, 0, True, False)
    halfstep(1, 1, True, False)

    def pair_body(kp, carry):
        k0 = 2 * kp
        halfstep(k0, 0, False, False)
        halfstep(k0 + 1, 1, False, False)
        return carry

    lax.fori_loop(1, NPAIR - 1, pair_body, 0)

    k0 = NCHUNK - 2
    halfstep(k0, 0, False, True)
    halfstep(k0 + 1, 1, False, True)

    outstream_wait(NCHUNK - 2, 0)
    outstream_wait(NCHUNK - 1, 1)


def kernel(mlmm_distances, atomic_charges, mlmm_atomic_charges,
           mlmm_idxu, mlmm_idxv, mlmm_vectors, atomic_dipoles):
    padn = PAD - N_ML

    def packw(a, b):
        a16 = lax.bitcast_convert_type(a.astype(jnp.bfloat16),
                                       jnp.uint16).astype(jnp.uint32)
        b16 = lax.bitcast_convert_type(b.astype(jnp.bfloat16),
                                       jnp.uint16).astype(jnp.uint32)
        return lax.bitcast_convert_type(a16 | (b16 << 16), jnp.int32)

    qdx = jnp.pad(packw(atomic_charges, atomic_dipoles[:, 0]), (0, padn))
    dydz = jnp.pad(packw(atomic_dipoles[:, 1], atomic_dipoles[:, 2]),
                   (0, padn))
    qmm = jnp.pad(mlmm_atomic_charges, (0, PAD - N_MM))
    vx = mlmm_vectors[:, 0]
    vy = mlmm_vectors[:, 1]
    vz = mlmm_vectors[:, 2]

    def dbuf(dt=jnp.float32):
        return pltpu.VMEM((2 * CHUNK,), dt)

    f = pl.kernel(
        _sc_body,
        out_type=jax.ShapeDtypeStruct((E,), jnp.float32),
        mesh=plsc.VectorSubcoreMesh(core_axis_name="c", subcore_axis_name="s"),
        scratch_types=[
            pltpu.VMEM_SHARED((PAD,), jnp.int32),
            pltpu.VMEM_SHARED((PAD,), jnp.int32),
            pltpu.VMEM_SHARED((PAD,), jnp.float32),
            dbuf(), dbuf(jnp.int32), dbuf(jnp.int32), dbuf(), dbuf(), dbuf(),
            dbuf(jnp.int32), dbuf(jnp.int32), dbuf(),
            dbuf(),
            pltpu.SemaphoreType.DMA, pltpu.SemaphoreType.DMA,
            pltpu.SemaphoreType.DMA, pltpu.SemaphoreType.DMA,
            pltpu.SemaphoreType.DMA, pltpu.SemaphoreType.DMA,
        ],
    )
    return f(mlmm_distances, mlmm_idxu, mlmm_idxv, vx, vy, vz,
             qdx, dydz, qmm)
